# baseline (speedup 1.0000x reference)
import functools

import jax
import jax.numpy as jnp
from jax.experimental import pallas as pl
from jax.experimental.pallas import tpu as pltpu

_K = 256
_ROWS = 16


def _sortable_i32(x):
    b = jax.lax.bitcast_convert_type(x, jnp.int32)
    return b ^ ((b >> 31) & jnp.int32(0x7FFFFFFF))


def _cellmax(x):
    v = x
    while v.shape[1] > 256:
        h = v.shape[1] // 2
        v = jnp.maximum(v[:, :h], v[:, h:])
    return v


def _tc_body(s_ref, p_ref, w_ref):
    s = s_ref[...]
    rows, n = s.shape

    m = jnp.max(s, axis=1, keepdims=True)
    e = jnp.exp(s - m)
    z = jnp.sum(e, axis=1, keepdims=True)
    p_ref[...] = e / z

    l = jnp.sign(s) * (1.0 - s)
    ml_ = jnp.max(l, axis=1, keepdims=True)

    kw = _sortable_i32(s)
    kl = _sortable_i32(l)

    kk = jnp.int32(_K)

    lbw = _sortable_i32(jnp.min(_cellmax(s), axis=1, keepdims=True))
    lbl = _sortable_i32(jnp.min(_cellmax(l), axis=1, keepdims=True))
    ubw = _sortable_i32(m) + 1
    ubl = _sortable_i32(ml_) + 1

    zero = jnp.zeros((rows, 1), jnp.int32)

    khw = (kw >> 16).astype(jnp.int16)
    khl = (kl >> 16).astype(jnp.int16)
    lo16w0 = lbw >> 16
    hi16w0 = ((ubw - 1) >> 16) + 1
    lo16l0 = lbl >> 16
    hi16l0 = ((ubl - 1) >> 16) + 1

    def _c16(x16):
        c = x16
        while c.shape[1] > 256:
            h = c.shape[1] // 2
            c = c[:, :h] + c[:, h:]
        return jnp.sum(c.astype(jnp.int32), axis=1, keepdims=True)

    cw16lo0 = _c16((khw >= lo16w0.astype(jnp.int16)).astype(jnp.int16))
    cl16lo0 = _c16((khl >= lo16l0.astype(jnp.int16)).astype(jnp.int16))

    def ccond(carry):
        lw16, hw16, ll16, hl16 = carry[0], carry[1], carry[2], carry[3]
        return jnp.any(hw16 > lw16 + 1) | jnp.any(hl16 > ll16 + 1)

    def cstep(carry):
        lw16, hw16, ll16, hl16, cwlo, cwhi, cllo, clhi = carry
        mw = (lw16 + hw16) >> 1
        ml2 = (ll16 + hl16) >> 1
        cw = _c16((khw >= mw.astype(jnp.int16)).astype(jnp.int16))
        cl = _c16((khl >= ml2.astype(jnp.int16)).astype(jnp.int16))
        pw = cw >= kk
        pl_ = cl >= kk
        dw = hw16 > lw16 + 1
        dl = hl16 > ll16 + 1
        lw16 = jnp.where(dw & pw, mw, lw16)
        hw16 = jnp.where(dw & ~pw, mw, hw16)
        cwlo = jnp.where(dw & pw, cw, cwlo)
        cwhi = jnp.where(dw & ~pw, cw, cwhi)
        ll16 = jnp.where(dl & pl_, ml2, ll16)
        hl16 = jnp.where(dl & ~pl_, ml2, hl16)
        cllo = jnp.where(dl & pl_, cl, cllo)
        clhi = jnp.where(dl & ~pl_, cl, clhi)
        return lw16, hw16, ll16, hl16, cwlo, cwhi, cllo, clhi

    vt16w, _, vt16l, _, cge16w, cgt16w, cge16l, cgt16l = jax.lax.while_loop(
        ccond, cstep,
        (lo16w0, hi16w0, lo16l0, hi16l0, cw16lo0, zero, cl16lo0, zero))

    eq16w = (khw == vt16w.astype(jnp.int16)).astype(jnp.int16)
    eq16l = (khl == vt16l.astype(jnp.int16)).astype(jnp.int16)
    klow_w = ((kw & 0xFFFF) ^ 0x8000).astype(jnp.int16)
    klow_l = ((kl & 0xFFFF) ^ 0x8000).astype(jnp.int16)
    needw = kk - cgt16w
    needl = kk - cgt16l

    def fcond(carry):
        lw16, hw16, ll16, hl16 = carry[0], carry[1], carry[2], carry[3]
        return jnp.any(hw16 > lw16 + 1) | jnp.any(hl16 > ll16 + 1)

    def fstep(carry):
        lw16, hw16, ll16, hl16, cwlo, cwhi, cllo, clhi = carry
        mw = (lw16 + hw16) >> 1
        ml2 = (ll16 + hl16) >> 1
        cw = _c16(jnp.where(klow_w >= mw.astype(jnp.int16), eq16w,
                            jnp.int16(0)))
        cl = _c16(jnp.where(klow_l >= ml2.astype(jnp.int16), eq16l,
                            jnp.int16(0)))
        pw = cw >= needw
        pl_ = cl >= needl
        dw = hw16 > lw16 + 1
        dl = hl16 > ll16 + 1
        lw16 = jnp.where(dw & pw, mw, lw16)
        hw16 = jnp.where(dw & ~pw, mw, hw16)
        cwlo = jnp.where(dw & pw, cw, cwlo)
        cwhi = jnp.where(dw & ~pw, cw, cwhi)
        ll16 = jnp.where(dl & pl_, ml2, ll16)
        hl16 = jnp.where(dl & ~pl_, ml2, hl16)
        cllo = jnp.where(dl & pl_, cl, cllo)
        clhi = jnp.where(dl & ~pl_, cl, clhi)
        return lw16, hw16, ll16, hl16, cwlo, cwhi, cllo, clhi

    nly = jnp.full((rows, 1), -32769, jnp.int32)
    nhy = jnp.full((rows, 1), 32768, jnp.int32)
    neqw = cge16w - cgt16w
    neql = cge16l - cgt16l
    flow, _, flol, _, cfw_lo, cfw_hi, cfl_lo, cfl_hi = jax.lax.while_loop(
        fcond, fstep, (nly, nhy, nly, nhy, neqw, zero, neql, zero))

    lw = (vt16w << 16) | ((flow & 0xFFFF) ^ 0x8000)
    ll = (vt16l << 16) | ((flol & 0xFFFF) ^ 0x8000)
    cgew = cgt16w + cfw_lo
    cgtw = cgt16w + cfw_hi
    cgel = cgt16l + cfl_lo
    cgtl = cgt16l + cfl_hi

    rw = kk - cgtw
    rl = kk - cgtl

    idx = jax.lax.broadcasted_iota(jnp.int32, (rows, n), 1)
    eqw = kw == lw
    eql = kl == ll

    needs = jnp.any(cgew > kk) | jnp.any(cgel > kk)

    li0 = jnp.full((rows, 1), -1, jnp.int32)
    hi0i = jnp.full((rows, 1), n - 1, jnp.int32)

    def icond(carry):
        step = carry[4]
        return needs & (step < 15)

    def istep(carry):
        liw, hiw, lil, hil, step = carry
        miw = (liw + hiw) >> 1
        mil = (lil + hil) >> 1
        cw = jnp.sum((eqw & (idx <= miw)).astype(jnp.int32), axis=1,
                     keepdims=True)
        cl = jnp.sum((eql & (idx <= mil)).astype(jnp.int32), axis=1,
                     keepdims=True)
        pw = cw >= rw
        pl_ = cl >= rl
        hiw = jnp.where(pw, miw, hiw)
        liw = jnp.where(pw, liw, miw)
        hil = jnp.where(pl_, mil, hil)
        lil = jnp.where(pl_, lil, mil)
        return liw, hiw, lil, hil, step + 1

    _, itw, _, itl, _ = jax.lax.while_loop(
        icond, istep, (li0, hi0i, li0, hi0i, jnp.int32(0)))

    maskw = (kw > lw) | (eqw & (idx <= itw))
    maskl = (kl > ll) | (eql & (idx <= itl))

    ew = jnp.where(maskw, e, 0.0)
    zw = jnp.sum(ew, axis=1, keepdims=True)
    w_ref[:, 0:n] = ew / zw

    el = jnp.where(maskl, jnp.exp(l - ml_), 0.0)
    zl = jnp.sum(el, axis=1, keepdims=True)
    w_ref[:, n:2 * n] = el / zl


@jax.jit
def kernel(scores):
    b, n = scores.shape
    grid = b // _ROWS
    p_out, w_out = pl.pallas_call(
        _tc_body,
        grid=(grid,),
        in_specs=[pl.BlockSpec((_ROWS, n), lambda i: (i, 0))],
        out_specs=[
            pl.BlockSpec((_ROWS, n), lambda i: (i, 0)),
            pl.BlockSpec((_ROWS, 2 * n), lambda i: (i, 0)),
        ],
        out_shape=[
            jax.ShapeDtypeStruct((b, n), jnp.float32),
            jax.ShapeDtypeStruct((b, 2 * n), jnp.float32),
        ],
        compiler_params=pltpu.CompilerParams(
            dimension_semantics=("parallel",),
        ),
    )(scores)
    rho = jnp.full((b,), 0.5, jnp.float32)
    return (w_out, rho, p_out)
